# Initial kernel scaffold; baseline (speedup 1.0000x reference)
import jax
import jax.numpy as jnp
from jax.experimental import pallas as pl


def kernel(X, vertex, edges, W, eps):
    raise NotImplementedError("write your pallas kernel here")


# speedup vs baseline: 4.5331x; 4.5331x over previous
import functools

import jax
import jax.numpy as jnp
from jax import lax
from jax.experimental import pallas as pl
from jax.experimental.pallas import tpu as pltpu
from jax.experimental.pallas import tpu_sc as plsc

N = 10000
E = 320000
M = 10000
D_IN = 128
D_OUT_TOTAL = 128
HALF = 64

NC = 2
NS = 16
CHUNK = 128
K = 160
SK = 32
EP_TILE = K * CHUNK
EP = EP_TILE * NS
RZ = 632
R_ACC = RZ * NS
TRASH = N
RW_TAIL = N - 15 * RZ


def _matmul_body(x_ref, w_ref, o_ref):
    o_ref[...] = jnp.dot(x_ref[...], w_ref[...],
                         preferred_element_type=jnp.float32)


def _matmul(x, w):
    blk = 400
    return pl.pallas_call(
        _matmul_body,
        grid=(N // blk,),
        in_specs=[
            pl.BlockSpec((blk, D_IN), lambda i: (i, 0)),
            pl.BlockSpec((D_IN, D_OUT_TOTAL), lambda i: (0, 0)),
        ],
        out_specs=pl.BlockSpec((blk, D_OUT_TOTAL), lambda i: (i, 0)),
        out_shape=jax.ShapeDtypeStruct((N, D_OUT_TOTAL), jnp.float32),
    )(x, w)


def _residual_body(eps_ref, xp_ref, xv_ref, o_ref):
    o_ref[...] = (1.0 + eps_ref[0]) * xp_ref[...] + xv_ref[...]


def _residual(xp, xv, eps):
    blk = 400
    return pl.pallas_call(
        _residual_body,
        grid=(N // blk,),
        in_specs=[
            pl.BlockSpec(memory_space=pltpu.SMEM),
            pl.BlockSpec((blk, D_OUT_TOTAL), lambda i: (i, 0)),
            pl.BlockSpec((blk, D_OUT_TOTAL), lambda i: (i, 0)),
        ],
        out_specs=pl.BlockSpec((blk, D_OUT_TOTAL), lambda i: (i, 0)),
        out_shape=jax.ShapeDtypeStruct((N, D_OUT_TOTAL), jnp.float32),
    )(eps, xp, xv)


def _sc_body(xp2, vg, ev, vx, zz, out, idx_g, idx_s, rows, xe_s, xv_s):
    c = lax.axis_index("c")
    s = lax.axis_index("s")

    z0 = s * RZ
    pltpu.sync_copy(zz.at[pl.ds(z0, RZ)], xe_s.at[pl.ds(z0, RZ)])
    pltpu.sync_copy(zz.at[pl.ds(z0, RZ)], xv_s.at[pl.ds(z0, RZ)])
    plsc.subcore_barrier()

    for g in range(K // SK):
        pltpu.sync_copy(vg.at[c, s, pl.ds(g * SK, SK)], idx_g)
        pltpu.sync_copy(ev.at[s, pl.ds(g * SK, SK)], idx_s)

        def p1(j):
            pltpu.sync_copy(xp2.at[idx_g.at[j]], rows)
            pltpu.sync_copy(rows, xe_s.at[idx_s.at[j]], add=True)
        pl.loop(0, SK)(p1)

    plsc.subcore_barrier()

    for g in range(K // SK):
        pltpu.sync_copy(ev.at[s, pl.ds(g * SK, SK)], idx_s)
        pltpu.sync_copy(vx.at[s, pl.ds(g * SK, SK)], idx_g)

        def p2(j):
            pltpu.sync_copy(xe_s.at[idx_s.at[j]], rows)
            pltpu.sync_copy(rows, xv_s.at[idx_g.at[j]], add=True)
        pl.loop(0, SK)(p2)

    plsc.subcore_barrier()

    r0 = s * RZ

    @pl.when(s < NS - 1)
    def _full_stripe():
        pltpu.sync_copy(xv_s.at[pl.ds(r0, RZ)], out.at[pl.ds(r0, RZ), c])

    @pl.when(s == NS - 1)
    def _tail_stripe():
        pltpu.sync_copy(xv_s.at[pl.ds(r0, RW_TAIL)],
                        out.at[pl.ds(r0, RW_TAIL), c])


@functools.partial(
    pl.kernel,
    out_type=jax.ShapeDtypeStruct((N, NC, HALF), jnp.float32),
    mesh=plsc.VectorSubcoreMesh(core_axis_name="c", subcore_axis_name="s",
                                num_cores=NC, num_subcores=NS),
    compiler_params=pltpu.CompilerParams(use_tc_tiling_on_sc=False),
    scratch_types=[
        pltpu.VMEM((SK, CHUNK), jnp.int32),
        pltpu.VMEM((SK, CHUNK), jnp.int32),
        pltpu.VMEM((CHUNK, HALF), jnp.float32),
        pltpu.VMEM_SHARED((R_ACC, HALF), jnp.float32),
        pltpu.VMEM_SHARED((R_ACC, HALF), jnp.float32),
    ],
)
def _sc_scatter_gather(xp2, vg, ev, vx, zz, out, *scratch):
    _sc_body(xp2, vg, ev, vx, zz, out, *scratch)


def kernel(X, vertex, edges, W, eps):
    vertex = vertex.astype(jnp.int32)
    edges = edges.astype(jnp.int32)

    xp = _matmul(X, W)
    xp2 = xp.reshape(2 * N, HALF)

    pad = EP - E
    v2 = 2 * vertex
    vg = jnp.stack([
        jnp.concatenate([v2, jnp.zeros((pad,), jnp.int32)]),
        jnp.concatenate([v2 + 1, jnp.zeros((pad,), jnp.int32)]),
    ]).reshape(NC, NS, K, CHUNK)
    ev = jnp.concatenate(
        [edges, jnp.full((pad,), TRASH, jnp.int32)]).reshape(NS, K, CHUNK)
    vx = jnp.concatenate(
        [vertex, jnp.full((pad,), TRASH, jnp.int32)]).reshape(NS, K, CHUNK)
    zz = jnp.zeros((R_ACC, HALF), jnp.float32)

    xv3 = _sc_scatter_gather(xp2, vg, ev, vx, zz)
    xv = xv3.reshape(N, D_OUT_TOTAL)

    return _residual(xp, xv, eps)


# speedup vs baseline: 5.4036x; 1.1920x over previous
import functools

import jax
import jax.numpy as jnp
from jax import lax
from jax.experimental import pallas as pl
from jax.experimental.pallas import tpu as pltpu
from jax.experimental.pallas import tpu_sc as plsc

N = 10000
E = 320000
M = 10000
D_IN = 128
D_OUT_TOTAL = 128
HALF = 64

NC = 2
NS = 16
CHUNK = 128
K = 160
SK = 32
EP_TILE = K * CHUNK
EP = EP_TILE * NS
RZ = 632
R_ACC = RZ * NS
TRASH = N
RW_TAIL = N - 15 * RZ


def _matmul_body(x_ref, w_ref, o_ref):
    o_ref[...] = jnp.dot(x_ref[...], w_ref[...],
                         preferred_element_type=jnp.float32)


def _matmul(x, w):
    blk = 400
    return pl.pallas_call(
        _matmul_body,
        grid=(N // blk,),
        in_specs=[
            pl.BlockSpec((blk, D_IN), lambda i: (i, 0)),
            pl.BlockSpec((D_IN, D_OUT_TOTAL), lambda i: (0, 0)),
        ],
        out_specs=pl.BlockSpec((blk, D_OUT_TOTAL), lambda i: (i, 0)),
        out_shape=jax.ShapeDtypeStruct((N, D_OUT_TOTAL), jnp.float32),
    )(x, w)


def _residual_body(eps_ref, xp_ref, xv_ref, o_ref):
    o_ref[...] = (1.0 + eps_ref[0]) * xp_ref[...] + xv_ref[...]


def _residual(xp, xv, eps):
    blk = 400
    return pl.pallas_call(
        _residual_body,
        grid=(N // blk,),
        in_specs=[
            pl.BlockSpec(memory_space=pltpu.SMEM),
            pl.BlockSpec((blk, D_OUT_TOTAL), lambda i: (i, 0)),
            pl.BlockSpec((blk, D_OUT_TOTAL), lambda i: (i, 0)),
        ],
        out_specs=pl.BlockSpec((blk, D_OUT_TOTAL), lambda i: (i, 0)),
        out_shape=jax.ShapeDtypeStruct((N, D_OUT_TOTAL), jnp.float32),
    )(eps, xp, xv)


def _phase(src, dst, gsrc, ssrc, idx_g, idx_s, r0, r1, r2, r3,
           sg0, sg1, ss0, ss1):
    def gath(t, buf, sem):
        return pltpu.async_copy(src.at[idx_g.at[t]], buf, sem)

    def scat(t, buf, sem):
        return pltpu.async_copy(buf, dst.at[idx_s.at[t]], sem, add=True)

    npair = SK // 2
    for g in range(K // SK):
        pltpu.sync_copy(gsrc(g), idx_g)
        pltpu.sync_copy(ssrc(g), idx_s)
        d0 = gath(0, r0, sg0)
        d1 = gath(1, r1, sg1)
        d0.wait()
        d1.wait()

        def body(m):
            t = 4 * m
            dg0 = gath(t + 2, r2, sg0)
            dg1 = gath(t + 3, r3, sg1)
            ds0 = scat(t, r0, ss0)
            ds1 = scat(t + 1, r1, ss1)
            dg0.wait(); dg1.wait(); ds0.wait(); ds1.wait()
            dg0 = gath(t + 4, r0, sg0)
            dg1 = gath(t + 5, r1, sg1)
            ds0 = scat(t + 2, r2, ss0)
            ds1 = scat(t + 3, r3, ss1)
            dg0.wait(); dg1.wait(); ds0.wait(); ds1.wait()
        pl.loop(0, npair // 2 - 1)(body)

        dg0 = gath(SK - 2, r2, sg0)
        dg1 = gath(SK - 1, r3, sg1)
        ds0 = scat(SK - 4, r0, ss0)
        ds1 = scat(SK - 3, r1, ss1)
        dg0.wait(); dg1.wait(); ds0.wait(); ds1.wait()
        ds0 = scat(SK - 2, r2, ss0)
        ds1 = scat(SK - 1, r3, ss1)
        ds0.wait(); ds1.wait()


def _sc_body(xp2, vg, ev, vx, zz, out, idx_g, idx_s, r0, r1, r2, r3,
             sg0, sg1, ss0, ss1, xe_s, xv_s):
    c = lax.axis_index("c")
    s = lax.axis_index("s")

    z0 = s * RZ
    pltpu.sync_copy(zz.at[pl.ds(z0, RZ)], xe_s.at[pl.ds(z0, RZ)])
    pltpu.sync_copy(zz.at[pl.ds(z0, RZ)], xv_s.at[pl.ds(z0, RZ)])
    plsc.subcore_barrier()

    _phase(xp2, xe_s,
           lambda g: vg.at[c, s, pl.ds(g * SK, SK)],
           lambda g: ev.at[s, pl.ds(g * SK, SK)],
           idx_g, idx_s, r0, r1, r2, r3, sg0, sg1, ss0, ss1)

    plsc.subcore_barrier()

    _phase(xe_s, xv_s,
           lambda g: ev.at[s, pl.ds(g * SK, SK)],
           lambda g: vx.at[s, pl.ds(g * SK, SK)],
           idx_g, idx_s, r0, r1, r2, r3, sg0, sg1, ss0, ss1)

    plsc.subcore_barrier()

    r0 = s * RZ

    @pl.when(s < NS - 1)
    def _full_stripe():
        pltpu.sync_copy(xv_s.at[pl.ds(r0, RZ)], out.at[pl.ds(r0, RZ), c])

    @pl.when(s == NS - 1)
    def _tail_stripe():
        pltpu.sync_copy(xv_s.at[pl.ds(r0, RW_TAIL)],
                        out.at[pl.ds(r0, RW_TAIL), c])


@functools.partial(
    pl.kernel,
    out_type=jax.ShapeDtypeStruct((N, NC, HALF), jnp.float32),
    mesh=plsc.VectorSubcoreMesh(core_axis_name="c", subcore_axis_name="s",
                                num_cores=NC, num_subcores=NS),
    compiler_params=pltpu.CompilerParams(use_tc_tiling_on_sc=False),
    scratch_types=[
        pltpu.VMEM((SK, CHUNK), jnp.int32),
        pltpu.VMEM((SK, CHUNK), jnp.int32),
        pltpu.VMEM((CHUNK, HALF), jnp.float32),
        pltpu.VMEM((CHUNK, HALF), jnp.float32),
        pltpu.VMEM((CHUNK, HALF), jnp.float32),
        pltpu.VMEM((CHUNK, HALF), jnp.float32),
        pltpu.SemaphoreType.DMA,
        pltpu.SemaphoreType.DMA,
        pltpu.SemaphoreType.DMA,
        pltpu.SemaphoreType.DMA,
        pltpu.VMEM_SHARED((R_ACC, HALF), jnp.float32),
        pltpu.VMEM_SHARED((R_ACC, HALF), jnp.float32),
    ],
)
def _sc_scatter_gather(xp2, vg, ev, vx, zz, out, *scratch):
    _sc_body(xp2, vg, ev, vx, zz, out, *scratch)


def kernel(X, vertex, edges, W, eps):
    vertex = vertex.astype(jnp.int32)
    edges = edges.astype(jnp.int32)

    xp = _matmul(X, W)
    xp2 = xp.reshape(2 * N, HALF)

    pad = EP - E
    v2 = 2 * vertex
    vg = jnp.stack([
        jnp.concatenate([v2, jnp.zeros((pad,), jnp.int32)]),
        jnp.concatenate([v2 + 1, jnp.zeros((pad,), jnp.int32)]),
    ]).reshape(NC, NS, K, CHUNK)
    ev = jnp.concatenate(
        [edges, jnp.full((pad,), TRASH, jnp.int32)]).reshape(NS, K, CHUNK)
    vx = jnp.concatenate(
        [vertex, jnp.full((pad,), TRASH, jnp.int32)]).reshape(NS, K, CHUNK)
    zz = jnp.zeros((R_ACC, HALF), jnp.float32)

    xv3 = _sc_scatter_gather(xp2, vg, ev, vx, zz)
    xv = xv3.reshape(N, D_OUT_TOTAL)

    return _residual(xp, xv, eps)
